# Initial kernel scaffold; baseline (speedup 1.0000x reference)
import jax
import jax.numpy as jnp
from jax.experimental import pallas as pl


def kernel(noisy_pc, clean_pc, W1, b1, W2, b2, S1, sb1, S2, sb2):
    raise NotImplementedError("write your pallas kernel here")


# speedup vs baseline: 30.0166x; 30.0166x over previous
import jax
import jax.numpy as jnp
from jax.experimental import pallas as pl
from jax.experimental.pallas import tpu as pltpu

_NUM_PTS = 128
_K_SAMPLE = 32
_K_SCORE = 4
_SIGMA = 0.01
_FEAT = 128
_BIG = 1e30
_IBIG = 1 << 30


def _prep_body(sidx_ref, noisyT_ref, W1_ref, b1_ref, W2_ref, b2_ref,
               S1a_ref, S1b_ref, sb1_ref, S2_ref, sb2_ref, F_ref, E_ref):
    n = noisyT_ref.shape[2]
    px = noisyT_ref[0, 0:1, :]
    py = noisyT_ref[0, 1:2, :]
    pz = noisyT_ref[0, 2:3, :]
    sidx = sidx_ref[...]
    col = jax.lax.broadcasted_iota(jnp.int32, (_NUM_PTS, n), 1)
    smask = col == sidx
    zero = jnp.float32(0.0)
    sx = jnp.sum(jnp.where(smask, px, zero), axis=1, keepdims=True)
    sy = jnp.sum(jnp.where(smask, py, zero), axis=1, keepdims=True)
    sz = jnp.sum(jnp.where(smask, pz, zero), axis=1, keepdims=True)

    h1 = jnp.maximum(sx * W1_ref[0:1, :] + sy * W1_ref[1:2, :]
                     + sz * W1_ref[2:3, :] + b1_ref[...], zero)
    feat = jnp.dot(h1, W2_ref[...], preferred_element_type=jnp.float32) + b2_ref[...]
    zS = jnp.dot(feat, S1b_ref[...], preferred_element_type=jnp.float32) + sb1_ref[...]

    d1 = (sx - px) ** 2 + (sy - py) ** 2 + (sz - pz) ** 2
    for k in range(_K_SAMPLE):
        m = jnp.min(d1, axis=1, keepdims=True)
        eq = d1 == m
        fi = jnp.min(jnp.where(eq, col, _IBIG), axis=1, keepdims=True)
        sel = col == fi
        d1 = jnp.where(sel, _BIG, d1)
        fx = jnp.sum(jnp.where(sel, px, zero), axis=1, keepdims=True)
        fy = jnp.sum(jnp.where(sel, py, zero), axis=1, keepdims=True)
        fz = jnp.sum(jnp.where(sel, pz, zero), axis=1, keepdims=True)
        F_ref[0, k * _NUM_PTS:(k + 1) * _NUM_PTS, :] = jnp.concatenate(
            [fx, fy, fz], axis=1)
        xh = (fx - sx) * S1a_ref[0:1, :] + (fy - sy) * S1a_ref[1:2, :] \
            + (fz - sz) * S1a_ref[2:3, :]
        h = jnp.maximum(xh + zS, zero)
        ek = jnp.dot(h, S2_ref[...], preferred_element_type=jnp.float32) + sb2_ref[...]
        E_ref[0, k * _NUM_PTS:(k + 1) * _NUM_PTS, :] = ek


def _knn2_body(F_ref, E_ref, cleanT_ref, out_ref):
    r = F_ref.shape[1]
    m_pts = cleanT_ref.shape[2]
    zero = jnp.float32(0.0)
    fx = F_ref[0, :, 0:1]
    fy = F_ref[0, :, 1:2]
    fz = F_ref[0, :, 2:3]
    cx = cleanT_ref[0, 0:1, :]
    cy = cleanT_ref[0, 1:2, :]
    cz = cleanT_ref[0, 2:3, :]
    d = (fx - cx) ** 2 + (fy - cy) ** 2 + (fz - cz) ** 2
    col = jax.lax.broadcasted_iota(jnp.int32, (r, m_pts), 1)
    nx = jnp.zeros((r, 1), jnp.float32)
    ny = jnp.zeros((r, 1), jnp.float32)
    nz = jnp.zeros((r, 1), jnp.float32)
    for _ in range(_K_SCORE):
        mn = jnp.min(d, axis=1, keepdims=True)
        eq = d == mn
        fi = jnp.min(jnp.where(eq, col, _IBIG), axis=1, keepdims=True)
        sel = col == fi
        d = jnp.where(sel, _BIG, d)
        nx = nx + jnp.sum(jnp.where(sel, cx, zero), axis=1, keepdims=True)
        ny = ny + jnp.sum(jnp.where(sel, cy, zero), axis=1, keepdims=True)
        nz = nz + jnp.sum(jnp.where(sel, cz, zero), axis=1, keepdims=True)
    inv = jnp.float32(1.0 / _K_SCORE)
    dx = E_ref[0, :, 0:1] - (nx * inv - fx)
    dy = E_ref[0, :, 1:2] - (ny * inv - fy)
    dz = E_ref[0, :, 2:3] - (nz * inv - fz)
    out_ref[0, 0, 0] = jnp.sum(dx * dx + dy * dy + dz * dz)


def kernel(noisy_pc, clean_pc, W1, b1, W2, b2, S1, sb1, S2, sb2):
    B, N, _ = noisy_pc.shape
    M = clean_pc.shape[1]
    Q = _NUM_PTS * _K_SAMPLE

    sidx = jax.random.permutation(jax.random.key(1), N)[:_NUM_PTS]
    sidx = sidx.astype(jnp.int32).reshape(_NUM_PTS, 1)
    noisyT = jnp.transpose(noisy_pc, (0, 2, 1))
    cleanT = jnp.transpose(clean_pc, (0, 2, 1))

    fixed = lambda *shape: pl.BlockSpec(shape, lambda b: (0,) * len(shape))
    F, E = pl.pallas_call(
        _prep_body,
        grid=(B,),
        in_specs=[
            fixed(_NUM_PTS, 1),
            pl.BlockSpec((1, 3, N), lambda b: (b, 0, 0)),
            fixed(3, _FEAT), fixed(1, _FEAT),
            fixed(_FEAT, _FEAT), fixed(1, _FEAT),
            fixed(3, _FEAT), fixed(_FEAT, _FEAT), fixed(1, _FEAT),
            fixed(_FEAT, 3), fixed(1, 3),
        ],
        out_specs=[
            pl.BlockSpec((1, Q, 3), lambda b: (b, 0, 0)),
            pl.BlockSpec((1, Q, 3), lambda b: (b, 0, 0)),
        ],
        out_shape=[
            jax.ShapeDtypeStruct((B, Q, 3), jnp.float32),
            jax.ShapeDtypeStruct((B, Q, 3), jnp.float32),
        ],
    )(sidx, noisyT, W1, b1.reshape(1, _FEAT), W2, b2.reshape(1, _FEAT),
      S1[:3], S1[3:], sb1.reshape(1, _FEAT), S2, sb2.reshape(1, 3))

    R = 256
    T = Q // R
    parts = pl.pallas_call(
        _knn2_body,
        grid=(B, T),
        in_specs=[
            pl.BlockSpec((1, R, 3), lambda b, t: (b, t, 0)),
            pl.BlockSpec((1, R, 3), lambda b, t: (b, t, 0)),
            pl.BlockSpec((1, 3, M), lambda b, t: (b, 0, 0)),
        ],
        out_specs=pl.BlockSpec((1, 1, 1), lambda b, t: (b * T + t, 0, 0),
                               memory_space=pltpu.SMEM),
        out_shape=jax.ShapeDtypeStruct((B * T, 1, 1), jnp.float32),
    )(F, E, cleanT)

    denom = B * _NUM_PTS * _K_SAMPLE
    return 0.5 * (1.0 / _SIGMA) * jnp.sum(parts) / denom
